# baseline (speedup 1.0000x reference)
import functools

import jax
import jax.numpy as jnp
from jax import lax
from jax.experimental import pallas as pl
from jax.experimental.pallas import tpu as pltpu
from jax.experimental.pallas import tpu_sc as plsc

B = 4096
L = 200
D = 64
LA = 128
LB = L - LA
NC = 2
NS = 16
NW = NC * NS
ROWS_PER_W = B // NW


def _body(x_hbm, table_hbm, out_hbm, refs):
    (idxa0, idxb0, idxa1, idxb1, la0, lb0, la1, lb1,
     pa0, pb0, pa1, pb1, rows0, rows1,
     out_v, t0_v, g0, g1, i0, i1) = refs
    wid = lax.axis_index("s") * NC + lax.axis_index("c")
    base = wid * ROWS_PER_W

    pltpu.sync_copy(table_hbm.at[pl.ds(0, 1)], t0_v)
    t0 = [t0_v[0, pl.ds(k * 16, 16)] for k in range(4)]

    lane = lax.iota(jnp.int32, 16)
    idxas = (idxa0, idxa1)
    idxbs = (idxb0, idxb1)
    las = (la0, la1)
    lbs = (lb0, lb1)
    pas = (pa0, pa1)
    pbs = (pb0, pb1)
    rows = (rows0, rows1)
    gsems = (g0, g1)
    isems = (i0, i1)

    def copy_idx(r, slot):
        off = (base + r) * L
        pltpu.async_copy(x_hbm.at[pl.ds(off, LA)], idxas[slot], isems[slot])
        pltpu.async_copy(
            x_hbm.at[pl.ds(off + LA, LB)], idxbs[slot], isems[slot]
        )

    def wait_idx(r, slot):
        off = (base + r) * L
        pltpu.make_async_copy(
            x_hbm.at[pl.ds(off, LA)], idxas[slot], isems[slot]
        ).wait()
        pltpu.make_async_copy(
            x_hbm.at[pl.ds(off + LA, LB)], idxbs[slot], isems[slot]
        ).wait()

    def shift_idx(slot):
        for kk in range(8):
            v = idxas[slot][pl.ds(kk * 16, 16)]
            las[slot][pl.ds(kk * 16, 16)] = v >> 1
            pas[slot][pl.ds(kk * 16, 16)] = v & 1
        for kk in range(4):
            v = idxbs[slot][pl.ds(kk * 16, 16)]
            lbs[slot][pl.ds(kk * 16, 16)] = v >> 1
            pbs[slot][pl.ds(kk * 16, 16)] = v & 1
        v = idxbs[slot][pl.ds(LB - 16, 16)]
        lbs[slot][pl.ds(LB - 16, 16)] = v >> 1
        pbs[slot][pl.ds(LB - 16, 16)] = v & 1

    def start_gather(slot):
        pltpu.async_copy(
            table_hbm.at[las[slot]], rows[slot].at[pl.ds(0, LA)], gsems[slot]
        )
        pltpu.async_copy(
            table_hbm.at[lbs[slot]], rows[slot].at[pl.ds(LA, LB)], gsems[slot]
        )

    def wait_gather(slot):
        pltpu.make_async_copy(
            table_hbm.at[las[slot]], rows[slot].at[pl.ds(0, LA)], gsems[slot]
        ).wait()
        pltpu.make_async_copy(
            table_hbm.at[lbs[slot]], rows[slot].at[pl.ds(LA, LB)], gsems[slot]
        ).wait()

    def count_npad(slot):
        npad_i = jnp.zeros((16,), jnp.int32)
        for kk in range(8):
            v = idxas[slot][pl.ds(kk * 16, 16)]
            npad_i = npad_i + plsc.all_reduce_population_count(v == 0)
        for kk in range(4):
            v = idxbs[slot][pl.ds(kk * 16, 16)]
            npad_i = npad_i + plsc.all_reduce_population_count(v == 0)
        vtail = idxbs[slot][pl.ds(LB - 16, 16)]
        npad_i = npad_i + plsc.all_reduce_population_count(
            (vtail == 0) & (lane >= 8)
        )
        npad_v = npad_i.astype(jnp.float32)
        return npad_v, 1.0 / jnp.maximum(float(L) - npad_v, 1.0)

    def accumulate(r, slot, npad_v, recip_v):
        buf = rows[slot]

        def acc16(j0, hv, acc, tstart=0):
            a0, a1, a2, a3 = acc
            for t in range(tstart, 16):
                j = j0 + t
                half = hv[t] * D
                a0 = a0 + buf[j, pl.ds(half, 16)]
                a1 = a1 + buf[j, pl.ds(half + 16, 16)]
                a2 = a2 + buf[j, pl.ds(half + 32, 16)]
                a3 = a3 + buf[j, pl.ds(half + 48, 16)]
            return (a0, a1, a2, a3)

        def acc_a(i, acc):
            hv = pas[slot][pl.ds(i * 16, 16)]
            return acc16(i * 16, hv, acc)

        def acc_b(i, acc):
            hv = pbs[slot][pl.ds(i * 16, 16)]
            return acc16(LA + i * 16, hv, acc)

        zero = jnp.zeros((16,), jnp.float32)
        accs = lax.fori_loop(0, LA // 16, acc_a, (zero, zero, zero, zero))
        accs = lax.fori_loop(0, LB // 16, acc_b, accs)
        hv = pbs[slot][pl.ds(LB - 16, 16)]
        accs = acc16(LA + LB - 16, hv, accs, tstart=8)
        for k in range(4):
            out_v[r, pl.ds(k * 16, 16)] = (accs[k] - npad_v * t0[k]) * recip_v

    def step(r, slot, prefetch):
        npad_v, recip_v = count_npad(slot)
        wait_gather(slot)
        if prefetch:
            copy_idx(r + 2, slot)
        wait_idx(r + 1, 1 - slot)
        shift_idx(1 - slot)
        start_gather(1 - slot)
        accumulate(r, slot, npad_v, recip_v)

    copy_idx(0, 0)
    wait_idx(0, 0)
    copy_idx(1, 1)
    shift_idx(0)
    start_gather(0)

    def pair_body(p, carry):
        step(2 * p, 0, prefetch=True)
        step(2 * p + 1, 1, prefetch=True)
        return carry

    lax.fori_loop(0, ROWS_PER_W // 2 - 1, pair_body, 0)

    r = ROWS_PER_W - 2
    step(r, 0, prefetch=False)
    npad_v, recip_v = count_npad(1)
    wait_gather(1)
    accumulate(r + 1, 1, npad_v, recip_v)

    pltpu.sync_copy(out_v, out_hbm.at[pl.ds(base, ROWS_PER_W)])


@functools.partial(
    pl.kernel,
    out_type=jax.ShapeDtypeStruct((B, D), jnp.float32),
    mesh=plsc.VectorSubcoreMesh(core_axis_name="c", subcore_axis_name="s"),
    compiler_params=pltpu.CompilerParams(needs_layout_passes=False),
    scratch_types=[
        pltpu.VMEM((LA,), jnp.int32),
        pltpu.VMEM((LB,), jnp.int32),
        pltpu.VMEM((LA,), jnp.int32),
        pltpu.VMEM((LB,), jnp.int32),
        pltpu.VMEM((LA,), jnp.int32),
        pltpu.VMEM((LB,), jnp.int32),
        pltpu.VMEM((LA,), jnp.int32),
        pltpu.VMEM((LB,), jnp.int32),
        pltpu.VMEM((LA,), jnp.int32),
        pltpu.VMEM((LB,), jnp.int32),
        pltpu.VMEM((LA,), jnp.int32),
        pltpu.VMEM((LB,), jnp.int32),
        pltpu.VMEM((L, 2 * D), jnp.float32),
        pltpu.VMEM((L, 2 * D), jnp.float32),
        pltpu.VMEM((ROWS_PER_W, D), jnp.float32),
        pltpu.VMEM((1, 2 * D), jnp.float32),
        pltpu.SemaphoreType.DMA,
        pltpu.SemaphoreType.DMA,
        pltpu.SemaphoreType.DMA,
        pltpu.SemaphoreType.DMA,
    ],
)
def _encoder_kernel(x_hbm, table_hbm, out_hbm, *refs):
    _body(x_hbm, table_hbm, out_hbm, refs)


def kernel(x, table):
    return _encoder_kernel(
        x.reshape(-1).astype(jnp.int32), table.reshape(-1, 2 * D)
    )


# speedup vs baseline: 1.1478x; 1.1478x over previous
import functools

import jax
import jax.numpy as jnp
from jax import lax
from jax.experimental import pallas as pl
from jax.experimental.pallas import tpu as pltpu
from jax.experimental.pallas import tpu_sc as plsc

B = 4096
L = 200
D = 64
NC = 2
NS = 16
NW = NC * NS
ROWS_PER_W = B // NW
CHUNK = 16
NCHUNK = ROWS_PER_W // CHUNK
UNROLL = 8


def _body(x_hbm, table_hbm, out_hbm, idx_v, rows_v, t0_v, out_v, sem0, sem1):
    wid = lax.axis_index("s") * NC + lax.axis_index("c")
    base = wid * ROWS_PER_W

    pltpu.sync_copy(table_hbm.at[pl.ds(0, 1)], t0_v)
    t0 = [t0_v[0, pl.ds(k * 16, 16)] for k in range(4)]

    sems = (sem0, sem1)

    def start_gather(r, slot):
        pltpu.async_copy(table_hbm.at[idx_v.at[r]], rows_v.at[slot], sems[slot])

    def wait_gather(r, slot):
        pltpu.make_async_copy(
            table_hbm.at[idx_v.at[r]], rows_v.at[slot], sems[slot]
        ).wait()

    lane = lax.iota(jnp.int32, 16)

    def chunk_body(c, carry):
        cbase = base + c * CHUNK
        pltpu.sync_copy(x_hbm.at[pl.ds(cbase, CHUNK)], idx_v)
        start_gather(0, 0)
        for r in range(CHUNK):
            slot = r % 2
            if r + 1 < CHUNK:
                start_gather(r + 1, (r + 1) % 2)

            npad_i = jnp.zeros((16,), jnp.int32)
            for kk in range(12):
                v = idx_v[r, pl.ds(kk * 16, 16)]
                npad_i = npad_i + plsc.all_reduce_population_count(v == 0)
            vtail = idx_v[r, pl.ds(L - 16, 16)]
            npad_i = npad_i + plsc.all_reduce_population_count(
                (vtail == 0) & (lane >= 8)
            )
            npad_v = npad_i.astype(jnp.float32)
            recip_v = 1.0 / jnp.maximum(float(L) - npad_v, 1.0)

            wait_gather(r, slot)

            def acc_body(i, acc):
                a0, a1, a2, a3 = acc
                for jj in range(UNROLL):
                    j = i * UNROLL + jj
                    a0 = a0 + rows_v[slot, j, pl.ds(0, 16)]
                    a1 = a1 + rows_v[slot, j, pl.ds(16, 16)]
                    a2 = a2 + rows_v[slot, j, pl.ds(32, 16)]
                    a3 = a3 + rows_v[slot, j, pl.ds(48, 16)]
                return (a0, a1, a2, a3)

            zero = jnp.zeros((16,), jnp.float32)
            accs = lax.fori_loop(0, L // UNROLL, acc_body, (zero, zero, zero, zero))
            for k in range(4):
                out_v[r, pl.ds(k * 16, 16)] = (accs[k] - npad_v * t0[k]) * recip_v
        pltpu.sync_copy(out_v, out_hbm.at[pl.ds(cbase, CHUNK)])
        return carry

    lax.fori_loop(0, NCHUNK, chunk_body, 0)


@functools.partial(
    pl.kernel,
    out_type=jax.ShapeDtypeStruct((B, D), jnp.float32),
    mesh=plsc.VectorSubcoreMesh(core_axis_name="c", subcore_axis_name="s"),
    compiler_params=pltpu.CompilerParams(
        needs_layout_passes=False, use_tc_tiling_on_sc=False
    ),
    scratch_types=[
        pltpu.VMEM((CHUNK, L), jnp.int32),
        pltpu.VMEM((2, L, D), jnp.float32),
        pltpu.VMEM((1, D), jnp.float32),
        pltpu.VMEM((CHUNK, D), jnp.float32),
        pltpu.SemaphoreType.DMA,
        pltpu.SemaphoreType.DMA,
    ],
)
def _encoder_kernel(x_hbm, table_hbm, out_hbm, idx_v, rows_v, t0_v, out_v, s0, s1):
    _body(x_hbm, table_hbm, out_hbm, idx_v, rows_v, t0_v, out_v, s0, s1)


def kernel(x, table):
    return _encoder_kernel(x.astype(jnp.int32), table)
